# baseline (speedup 1.0000x reference)
import functools

import jax
import jax.numpy as jnp
from jax import lax
from jax.experimental import pallas as pl
from jax.experimental.pallas import tpu as pltpu
from jax.experimental.pallas import tpu_sc as plsc

_NC = 2
_NS = 16
_NW = _NC * _NS

_B = 2
_NPTS = 16384
_TOT = _B * _NPTS
_PW = _TOT // _NW
_P = 32
_NCHUNK = _PW // _P
_ROWS = 8 * _P
_NJ = _ROWS // 128

_SA = 32
_CA = 64
_SB = 16
_CB = 128
_COUT = _CA + _CB


def _corner_data(x, y, z, scale, hi, stride_x, stride_y, bbase):
    def axis(v):
        s = jnp.minimum(jnp.maximum(v * scale, 0.01), hi)
        lo = s.astype(jnp.int32)
        frac = s - lo.astype(jnp.float32)
        hi_i = jnp.where(frac > 0.0, lo + 1, lo)
        return lo, hi_i, hi_i.astype(jnp.float32) - s, frac

    x1, x2, wx1, wx2 = axis(x)
    y1, y2, wy1, wy2 = axis(y)
    z1, z2, wz1, wz2 = axis(z)
    ix1 = x1 * stride_x + bbase
    ix2 = x2 * stride_x + bbase
    iy1 = y1 * stride_y
    iy2 = y2 * stride_y
    idxs = []
    ws = []
    for xi, wxs in ((ix1, wx1), (ix2, wx2)):
        for yi, wys in ((iy1, wy1), (iy2, wy2)):
            wxy = wxs * wys
            base_xy = xi + yi
            for zi, wzs in ((z1, wz1), (z2, wz2)):
                idxs.append(base_xy + zi)
                ws.append(wxy * wzs)
    return idxs, ws


def _sc_body(tab_a, tab_b, coords, out, coords_v, idx_a, idx_b, w_a, w_b,
             rows_a, rows_b, out_v, spmem_b, sem0, sem1, osem0, osem1):
    cid = lax.axis_index("c")
    sid = lax.axis_index("s")
    wid = sid * _NC + cid
    b = wid // (_NW // _B)
    m0 = lax.rem(wid, _NW // _B) * _PW
    base_pt = b * _NPTS + m0

    del spmem_b
    pltpu.sync_copy(coords.at[pl.ds(base_pt * 3, _PW * 3)], coords_v)

    iota = lax.iota(jnp.int32, 16)
    iota3 = iota * 3
    base_a = b * (_SA * _SA * _SA)
    base_b = b * (_SB * _SB * _SB)
    sems = (sem0, sem1)

    def compute_and_fire(ci, buf):
        sem = sems[buf]
        for g in range(_P // 16):
            off = ci * (_P * 3) + g * 48
            x = plsc.load_gather(coords_v, [iota3 + off])
            y = plsc.load_gather(coords_v, [iota3 + (off + 1)])
            z = plsc.load_gather(coords_v, [iota3 + (off + 2)])
            for (idx_ref, w_ref, idxs, ws) in (
                (idx_a, w_a) + _corner_data(x, y, z, float(_SA), _SA - 1.01,
                                            _SA * _SA, _SA, base_a),
                (idx_b, w_b) + _corner_data(x, y, z, float(_SB), _SB - 1.01,
                                            _SB * _SB, _SB, base_b),
            ):
                for r in range(8):
                    slot0 = r * _P + g * 16
                    idx_ref[buf, slot0 >> 7, pl.ds(slot0 & 127, 16)] = idxs[r]
                    w_ref[buf, r, pl.ds(g * 16, 16)] = ws[r]
        for j in range(_NJ):
            pltpu.async_copy(tab_a.at[idx_a.at[buf, j]],
                             rows_a.at[buf, pl.ds(j * 128, 128)], sem)
        for j in range(_NJ):
            pltpu.async_copy(tab_b.at[idx_b.at[buf, j]],
                             rows_b.at[buf, pl.ds(j * 128, 128)], sem)

    def drain(buf):
        sem = sems[buf]
        for j in range(_NJ):
            pltpu.make_async_copy(tab_a.at[idx_a.at[buf, j]],
                                  rows_a.at[buf, pl.ds(j * 128, 128)],
                                  sem).wait()
        for j in range(_NJ):
            pltpu.make_async_copy(tab_b.at[idx_b.at[buf, j]],
                                  rows_b.at[buf, pl.ds(j * 128, 128)],
                                  sem).wait()

    def combine_group(buf, g):
        ws_a = [w_a[buf, r, pl.ds(g * 16, 16)] for r in range(8)]
        ws_b = [w_b[buf, r, pl.ds(g * 16, 16)] for r in range(8)]

        @plsc.parallel_loop(0, 16, 1, unroll=4)
        def pbody(t):
            for dp in range(1):
                p = t + dp + g * 16
                pv = jnp.full((16,), p, dtype=jnp.int32)
                spl_a = [jnp.take_along_axis(w, pv, axis=0,
                                             mode="promise_in_bounds")
                         for w in ws_a]
                spl_b = [jnp.take_along_axis(w, pv, axis=0,
                                             mode="promise_in_bounds")
                         for w in ws_b]

                for rows, spl, col0, c_dim in ((rows_a, spl_a, 0, _CA),
                                               (rows_b, spl_b, _CA, _CB)):
                    for cb in range(c_dim // 16):
                        cs = cb * 16
                        vals = [spl[r] * rows[buf, r * _P + p, pl.ds(cs, 16)]
                                for r in range(8)]
                        acc = ((vals[0] + vals[1]) + (vals[2] + vals[3])) + \
                              ((vals[4] + vals[5]) + (vals[6] + vals[7]))
                        out_v[buf, p, pl.ds(col0 + cs, 16)] = acc

    osems = (osem0, osem1)

    def combine(ci, buf):
        @pl.when(ci >= 2)
        def _():
            pltpu.make_async_copy(
                out_v.at[buf], out.at[pl.ds(base_pt + (ci - 2) * _P, _P)],
                osems[buf]).wait()

        for g in range(_P // 16):
            combine_group(buf, g)
        pltpu.async_copy(out_v.at[buf],
                         out.at[pl.ds(base_pt + ci * _P, _P)], osems[buf])

    compute_and_fire(0, 0)
    compute_and_fire(1, 1)

    def pair(ci2, carry):
        base = ci2 * 2
        drain(0)
        combine(base, 0)

        @pl.when(base + 2 < _NCHUNK)
        def _():
            compute_and_fire(base + 2, 0)

        drain(1)
        combine(base + 1, 1)

        @pl.when(base + 3 < _NCHUNK)
        def _():
            compute_and_fire(base + 3, 1)

        return carry

    lax.fori_loop(0, _NCHUNK // 2, pair, 0)

    for buf, ci in ((0, _NCHUNK - 2), (1, _NCHUNK - 1)):
        pltpu.make_async_copy(
            out_v.at[buf], out.at[pl.ds(base_pt + ci * _P, _P)],
            osems[buf]).wait()


_mesh = plsc.VectorSubcoreMesh(core_axis_name="c", subcore_axis_name="s",
                               num_cores=_NC, num_subcores=_NS)

_proj = functools.partial(
    pl.kernel,
    out_type=jax.ShapeDtypeStruct((_TOT, _COUT), jnp.float32),
    mesh=_mesh,
    compiler_params=pltpu.CompilerParams(needs_layout_passes=False,
                                         use_tc_tiling_on_sc=False),
    scratch_types=[
        pltpu.VMEM((_PW * 3,), jnp.float32),
        pltpu.VMEM((2, _NJ, 128), jnp.int32),
        pltpu.VMEM((2, _NJ, 128), jnp.int32),
        pltpu.VMEM((2, 8, _P), jnp.float32),
        pltpu.VMEM((2, 8, _P), jnp.float32),
        pltpu.VMEM((2, _ROWS, _CA), jnp.float32),
        pltpu.VMEM((2, _ROWS, _CB), jnp.float32),
        pltpu.VMEM((2, _P, _COUT), jnp.float32),
        pltpu.VMEM((1,), jnp.float32),
        pltpu.SemaphoreType.DMA,
        pltpu.SemaphoreType.DMA,
        pltpu.SemaphoreType.DMA,
        pltpu.SemaphoreType.DMA,
    ],
)(_sc_body)


@jax.jit
def kernel(features0, features1, features2, features3, features4, mesh_coords):
    del features0, features3, features4
    tab_a = features1.reshape(_B * _SA * _SA * _SA, _CA)
    tab_b = features2.reshape(_B * _SB * _SB * _SB, _CB)
    coords = mesh_coords.reshape(-1)
    out = _proj(tab_a, tab_b, coords)
    return out.reshape(_B, _NPTS // 2, _COUT * 2)


# speedup vs baseline: 1.0401x; 1.0401x over previous
import functools

import jax
import jax.numpy as jnp
from jax import lax
from jax.experimental import pallas as pl
from jax.experimental.pallas import tpu as pltpu
from jax.experimental.pallas import tpu_sc as plsc

_NC = 2
_NS = 16
_NW = _NC * _NS

_B = 2
_NPTS = 16384
_TOT = _B * _NPTS
_PW = _TOT // _NW
_P = 16
_NCHUNK = _PW // _P
_ROWS = 8 * _P
_NJ = _ROWS // 128

_SA = 32
_CA = 64
_SB = 16
_CB = 128
_COUT = _CA + _CB


def _corner_data(x, y, z, scale, hi, stride_x, stride_y, bbase):
    def axis(v):
        s = jnp.minimum(jnp.maximum(v * scale, 0.01), hi)
        lo = s.astype(jnp.int32)
        frac = s - lo.astype(jnp.float32)
        hi_i = jnp.where(frac > 0.0, lo + 1, lo)
        return lo, hi_i, hi_i.astype(jnp.float32) - s, frac

    x1, x2, wx1, wx2 = axis(x)
    y1, y2, wy1, wy2 = axis(y)
    z1, z2, wz1, wz2 = axis(z)
    ix1 = x1 * stride_x + bbase
    ix2 = x2 * stride_x + bbase
    iy1 = y1 * stride_y
    iy2 = y2 * stride_y
    idxs = []
    ws = []
    for xi, wxs in ((ix1, wx1), (ix2, wx2)):
        for yi, wys in ((iy1, wy1), (iy2, wy2)):
            wxy = wxs * wys
            base_xy = xi + yi
            for zi, wzs in ((z1, wz1), (z2, wz2)):
                idxs.append(base_xy + zi)
                ws.append(wxy * wzs)
    return idxs, ws


def _sc_body(tab_a, tab_b, coords, out, coords_v, idx_a, idx_b, w_a, w_b,
             rows_a, rows_b, out_v, spmem_b, sem0, sem1, osem0, osem1):
    cid = lax.axis_index("c")
    sid = lax.axis_index("s")
    wid = sid * _NC + cid
    b = wid // (_NW // _B)
    m0 = lax.rem(wid, _NW // _B) * _PW
    base_pt = b * _NPTS + m0

    del spmem_b
    pltpu.sync_copy(coords.at[pl.ds(base_pt * 3, _PW * 3)], coords_v)

    iota = lax.iota(jnp.int32, 16)
    iota3 = iota * 3
    base_a = b * (_SA * _SA * _SA)
    base_b = b * (_SB * _SB * _SB)
    sems = (sem0, sem1)

    def compute_and_fire(ci, buf):
        sem = sems[buf]
        for g in range(_P // 16):
            off = ci * (_P * 3) + g * 48
            x = plsc.load_gather(coords_v, [iota3 + off])
            y = plsc.load_gather(coords_v, [iota3 + (off + 1)])
            z = plsc.load_gather(coords_v, [iota3 + (off + 2)])
            for (idx_ref, w_ref, idxs, ws) in (
                (idx_a, w_a) + _corner_data(x, y, z, float(_SA), _SA - 1.01,
                                            _SA * _SA, _SA, base_a),
                (idx_b, w_b) + _corner_data(x, y, z, float(_SB), _SB - 1.01,
                                            _SB * _SB, _SB, base_b),
            ):
                for r in range(8):
                    slot0 = r * _P + g * 16
                    idx_ref[buf, slot0 >> 7, pl.ds(slot0 & 127, 16)] = idxs[r]
                    w_ref[buf, r, pl.ds(g * 16, 16)] = ws[r]
        for j in range(_NJ):
            pltpu.async_copy(tab_a.at[idx_a.at[buf, j]],
                             rows_a.at[buf, pl.ds(j * 128, 128)], sem)
        for j in range(_NJ):
            pltpu.async_copy(tab_b.at[idx_b.at[buf, j]],
                             rows_b.at[buf, pl.ds(j * 128, 128)], sem)

    def drain(buf):
        sem = sems[buf]
        for j in range(_NJ):
            pltpu.make_async_copy(tab_a.at[idx_a.at[buf, j]],
                                  rows_a.at[buf, pl.ds(j * 128, 128)],
                                  sem).wait()
        for j in range(_NJ):
            pltpu.make_async_copy(tab_b.at[idx_b.at[buf, j]],
                                  rows_b.at[buf, pl.ds(j * 128, 128)],
                                  sem).wait()

    def combine_group(buf, g):
        ws_a = [w_a[buf, r, pl.ds(g * 16, 16)] for r in range(8)]
        ws_b = [w_b[buf, r, pl.ds(g * 16, 16)] for r in range(8)]

        @plsc.parallel_loop(0, 16, 1, unroll=4)
        def pbody(t):
            for dp in range(1):
                p = t + dp + g * 16
                pv = jnp.full((16,), p, dtype=jnp.int32)
                spl_a = [jnp.take_along_axis(w, pv, axis=0,
                                             mode="promise_in_bounds")
                         for w in ws_a]
                spl_b = [jnp.take_along_axis(w, pv, axis=0,
                                             mode="promise_in_bounds")
                         for w in ws_b]

                for rows, spl, col0, c_dim in ((rows_a, spl_a, 0, _CA),
                                               (rows_b, spl_b, _CA, _CB)):
                    for cb in range(c_dim // 16):
                        cs = cb * 16
                        vals = [spl[r] * rows[buf, r * _P + p, pl.ds(cs, 16)]
                                for r in range(8)]
                        acc = ((vals[0] + vals[1]) + (vals[2] + vals[3])) + \
                              ((vals[4] + vals[5]) + (vals[6] + vals[7]))
                        out_v[buf, p, pl.ds(col0 + cs, 16)] = acc

    osems = (osem0, osem1)

    def combine(ci, buf):
        @pl.when(ci >= 2)
        def _():
            pltpu.make_async_copy(
                out_v.at[buf], out.at[pl.ds(base_pt + (ci - 2) * _P, _P)],
                osems[buf]).wait()

        for g in range(_P // 16):
            combine_group(buf, g)
        pltpu.async_copy(out_v.at[buf],
                         out.at[pl.ds(base_pt + ci * _P, _P)], osems[buf])

    compute_and_fire(0, 0)
    compute_and_fire(1, 1)

    def pair(ci2, carry):
        base = ci2 * 2
        drain(0)
        combine(base, 0)

        @pl.when(base + 2 < _NCHUNK)
        def _():
            compute_and_fire(base + 2, 0)

        drain(1)
        combine(base + 1, 1)

        @pl.when(base + 3 < _NCHUNK)
        def _():
            compute_and_fire(base + 3, 1)

        return carry

    lax.fori_loop(0, _NCHUNK // 2, pair, 0)

    for buf, ci in ((0, _NCHUNK - 2), (1, _NCHUNK - 1)):
        pltpu.make_async_copy(
            out_v.at[buf], out.at[pl.ds(base_pt + ci * _P, _P)],
            osems[buf]).wait()


_mesh = plsc.VectorSubcoreMesh(core_axis_name="c", subcore_axis_name="s",
                               num_cores=_NC, num_subcores=_NS)

_proj = functools.partial(
    pl.kernel,
    out_type=jax.ShapeDtypeStruct((_TOT, _COUT), jnp.float32),
    mesh=_mesh,
    compiler_params=pltpu.CompilerParams(needs_layout_passes=False,
                                         use_tc_tiling_on_sc=False),
    scratch_types=[
        pltpu.VMEM((_PW * 3,), jnp.float32),
        pltpu.VMEM((2, _NJ, 128), jnp.int32),
        pltpu.VMEM((2, _NJ, 128), jnp.int32),
        pltpu.VMEM((2, 8, _P), jnp.float32),
        pltpu.VMEM((2, 8, _P), jnp.float32),
        pltpu.VMEM((2, _ROWS, _CA), jnp.float32),
        pltpu.VMEM((2, _ROWS, _CB), jnp.float32),
        pltpu.VMEM((2, _P, _COUT), jnp.float32),
        pltpu.VMEM((1,), jnp.float32),
        pltpu.SemaphoreType.DMA,
        pltpu.SemaphoreType.DMA,
        pltpu.SemaphoreType.DMA,
        pltpu.SemaphoreType.DMA,
    ],
)(_sc_body)


@jax.jit
def kernel(features0, features1, features2, features3, features4, mesh_coords):
    del features0, features3, features4
    tab_a = features1.reshape(_B * _SA * _SA * _SA, _CA)
    tab_b = features2.reshape(_B * _SB * _SB * _SB, _CB)
    coords = mesh_coords.reshape(-1)
    out = _proj(tab_a, tab_b, coords)
    return out.reshape(_B, _NPTS // 2, _COUT * 2)
